# baseline (speedup 1.0000x reference)
import functools

import jax
import jax.numpy as jnp
from jax import lax
from jax.experimental import pallas as pl
from jax.experimental.pallas import tpu as pltpu
from jax.experimental.pallas import tpu_sc as plsc

NC = 2
NS = 16
NW = NC * NS


def _make_edge_agg(N, NG, D, G, nb):
    base_g = NG // NW
    rem = NG % NW
    peel = (nb - 1) + (base_g - nb) % nb
    assert base_g >= peel + 2
    rows_per_sub = N // NS
    mesh = plsc.VectorSubcoreMesh(core_axis_name="c", subcore_axis_name="s")

    @functools.partial(
        pl.kernel,
        out_type=jax.ShapeDtypeStruct((NC, N, D), jnp.float32),
        mesh=mesh,
        compiler_params=pltpu.CompilerParams(use_tc_tiling_on_sc=False),
        scratch_types=[
            pltpu.VMEM_SHARED((N, D), jnp.float32),
            pltpu.VMEM((base_g + 1, G), jnp.int32),
            pltpu.VMEM((base_g + 1, G), jnp.int32),
        ] + [pltpu.VMEM((G, D), jnp.float32)] * nb
          + [pltpu.SemaphoreType.DMA] * (2 * nb),
    )
    def agg_kernel(table_hbm, src_hbm, dst_hbm, zeros_hbm, out_hbm,
                   acc, isl, idl, *bufs):
        rows = bufs[:nb]
        sg = bufs[nb:2 * nb]
        ss = bufs[2 * nb:3 * nb]
        cid = lax.axis_index("c")
        sid = lax.axis_index("s")
        wid = sid * NC + cid
        r0 = sid * rows_per_sub
        row0 = base_g * wid + jnp.minimum(wid, rem)
        pltpu.async_copy(zeros_hbm, acc.at[pl.ds(r0, rows_per_sub)], sg[0])
        pltpu.async_copy(src_hbm.at[pl.ds(row0, base_g + 1)], isl, sg[1])
        pltpu.async_copy(dst_hbm.at[pl.ds(row0, base_g + 1)], idl, ss[0])
        pltpu.make_async_copy(zeros_hbm, acc.at[pl.ds(r0, rows_per_sub)], sg[0]).wait()
        pltpu.make_async_copy(src_hbm.at[pl.ds(row0, base_g + 1)], isl, sg[1]).wait()
        pltpu.make_async_copy(dst_hbm.at[pl.ds(row0, base_g + 1)], idl, ss[0]).wait()
        has_extra = wid < rem
        plsc.subcore_barrier()

        def start_gather(j, b):
            pltpu.async_copy(table_hbm.at[isl.at[j]], rows[b], sg[b])

        def wait_gather(j, b):
            pltpu.make_async_copy(table_hbm.at[isl.at[j]], rows[b], sg[b]).wait()

        def start_scatter(j, b):
            pltpu.async_copy(rows[b], acc.at[idl.at[j]], ss[b], add=True)

        def wait_scatter(j, b):
            pltpu.make_async_copy(rows[b], acc.at[idl.at[j]], ss[b]).wait()

        start_gather(0, 0)
        for g in range(peel):
            wait_gather(g, g % nb)
            if g >= nb - 1:
                wait_scatter(g - (nb - 1), (g + 1) % nb)
            start_gather(g + 1, (g + 1) % nb)
            start_scatter(g, g % nb)

        @pl.loop(0, (base_g - 1 - peel) // nb)
        def _(i):
            for k in range(nb):
                g = nb * i + peel + k
                b = (peel + k) % nb
                wait_gather(g, b)
                wait_scatter(g - (nb - 1), (b + 1) % nb)
                start_gather(g + 1, (b + 1) % nb)
                start_scatter(g, b)

        g = base_g - 1
        b = g % nb
        bx = (b + 1) % nb
        wait_gather(g, b)
        wait_scatter(g - (nb - 1), bx)

        @pl.when(has_extra)
        def _():
            start_gather(base_g, bx)
        start_scatter(g, b)

        @pl.when(has_extra)
        def _():
            wait_gather(base_g, bx)
            start_scatter(base_g, bx)

        for g2 in range(base_g - nb + 1, base_g):
            wait_scatter(g2, g2 % nb)

        @pl.when(has_extra)
        def _():
            wait_scatter(base_g, base_g % nb)

        plsc.subcore_barrier()
        pltpu.sync_copy(acc.at[pl.ds(r0, rows_per_sub)],
                        out_hbm.at[cid, pl.ds(r0, rows_per_sub)])

    return agg_kernel


def _prep_call(x, D1, BLK):
    N, F = x.shape
    grid = N // BLK

    def body(x_ref, o_ref):
        o_ref[...] = jnp.concatenate(
            [x_ref[...], jnp.ones((BLK, 1), jnp.float32),
             jnp.zeros((BLK, D1 - F - 1), jnp.float32)], axis=1)

    return pl.pallas_call(
        body,
        grid=(grid,),
        in_specs=[pl.BlockSpec((BLK, F), lambda i: (i, 0))],
        out_specs=pl.BlockSpec((BLK, D1), lambda i: (i, 0)),
        out_shape=jax.ShapeDtypeStruct((N, D1), jnp.float32),
    )(x)


def _dense1_call(parts, x_ext, F, wl1t, b1, wr1t, wl2t, b2, wr2t, D1, D2, BLK):
    N = x_ext.shape[0]
    H = wl1t.shape[1]
    C = wl2t.shape[1]
    grid = N // BLK

    def body(p_ref, x_ref, wl1_ref, b1_ref, wr1_ref, wl2_ref, b2_ref, wr2_ref,
             y2_ref, zr_ref, dinv_ref):
        p = p_ref[0] + p_ref[1]
        feat = p[:, :F]
        deg = p[:, F:F + 1]
        dinv = 1.0 / jnp.maximum(deg, 1.0)
        mean = feat * dinv
        h = jnp.dot(mean, wl1_ref[...], preferred_element_type=jnp.float32)
        h = h + b1_ref[...]
        h = h + jnp.dot(x_ref[:, :F], wr1_ref[...], preferred_element_type=jnp.float32)
        h = jnp.maximum(h, 0.0)
        y2 = jnp.dot(h, wl2_ref[...], preferred_element_type=jnp.float32)
        zr = jnp.dot(h, wr2_ref[...], preferred_element_type=jnp.float32) + b2_ref[...]
        y2_ref[...] = jnp.pad(y2, ((0, 0), (0, D2 - C)))
        zr_ref[...] = zr
        dinv_ref[...] = jnp.broadcast_to(dinv, (BLK, 8))

    full = lambda s: pl.BlockSpec(s, lambda i: (0,) * len(s))
    return pl.pallas_call(
        body,
        grid=(grid,),
        in_specs=[
            pl.BlockSpec((NC, BLK, D1), lambda i: (0, i, 0)),
            pl.BlockSpec((BLK, D1), lambda i: (i, 0)),
            full((F, H)),
            full((1, H)),
            full((F, H)),
            full((H, C)),
            full((1, C)),
            full((H, C)),
        ],
        out_specs=[
            pl.BlockSpec((BLK, D2), lambda i: (i, 0)),
            pl.BlockSpec((BLK, C), lambda i: (i, 0)),
            pl.BlockSpec((BLK, 8), lambda i: (i, 0)),
        ],
        out_shape=[
            jax.ShapeDtypeStruct((N, D2), jnp.float32),
            jax.ShapeDtypeStruct((N, C), jnp.float32),
            jax.ShapeDtypeStruct((N, 8), jnp.float32),
        ],
    )(parts, x_ext, wl1t, b1, wr1t, wl2t, b2, wr2t)


def _dense2_call(parts2, zr, dinv, D2, BLK):
    N, C = zr.shape
    grid = N // BLK

    def body(p_ref, zr_ref, dinv_ref, o_ref):
        p = p_ref[0] + p_ref[1]
        v = p[:, :C] * dinv_ref[:, 0:1] + zr_ref[...]
        m = jnp.max(v, axis=1, keepdims=True)
        e = jnp.exp(v - m)
        s = jnp.sum(e, axis=1, keepdims=True)
        o_ref[...] = v - m - jnp.log(s)

    return pl.pallas_call(
        body,
        grid=(grid,),
        in_specs=[
            pl.BlockSpec((NC, BLK, D2), lambda i: (0, i, 0)),
            pl.BlockSpec((BLK, C), lambda i: (i, 0)),
            pl.BlockSpec((BLK, 8), lambda i: (i, 0)),
        ],
        out_specs=pl.BlockSpec((BLK, C), lambda i: (i, 0)),
        out_shape=jax.ShapeDtypeStruct((N, C), jnp.float32),
    )(parts2, zr, dinv)


def kernel(x, edge_index, W_l1, b_l1, W_r1, W_l2, b_l2, W_r2):
    N, F = x.shape
    E = edge_index.shape[1]
    C = W_l2.shape[0]
    D1 = F + 8
    D2 = 48
    BLK = 1000
    NP = ((N + 8 * NS - 1) // (8 * NS)) * (8 * NS)
    G1, G2 = 80, 128
    assert E % G1 == 0 and E % G2 == 0
    NG1, NG2 = E // G1, E // G2

    def idx_blocks(G, NG):
        ngp = ((NG + 1 + 7) // 8) * 8
        s = jnp.pad(edge_index[0], (0, ngp * G - E)).reshape(ngp, G)
        d = jnp.pad(edge_index[1], (0, ngp * G - E)).reshape(ngp, G)
        return s, d

    src1, dst1 = idx_blocks(G1, NG1)
    src2, dst2 = (src1, dst1) if G2 == G1 else idx_blocks(G2, NG2)

    x_ext = _prep_call(x, D1, BLK)
    zeros1 = jnp.zeros((NP // NS, D1), jnp.float32)
    zeros2 = jnp.zeros((NP // NS, D2), jnp.float32)

    agg1 = _make_edge_agg(NP, NG1, D1, G1, 2)
    parts1 = agg1(x_ext, src1, dst1, zeros1)

    y2, zr, dinv = _dense1_call(
        parts1, x_ext, F, W_l1.T, b_l1.reshape(1, -1), W_r1.T,
        W_l2.T, b_l2.reshape(1, -1), W_r2.T, D1, D2, BLK)

    agg2 = _make_edge_agg(NP, NG2, D2, G2, 3)
    parts2 = agg2(y2, src2, dst2, zeros2)

    return _dense2_call(parts2, zr, dinv, D2, BLK)


# speedup vs baseline: 1.0197x; 1.0197x over previous
import functools

import jax
import jax.numpy as jnp
from jax import lax
from jax.experimental import pallas as pl
from jax.experimental.pallas import tpu as pltpu
from jax.experimental.pallas import tpu_sc as plsc

NC = 2
NS = 16
NW = NC * NS


def _make_edge_agg(N, NG, D, G, nb):
    base_g = NG // NW
    rem = NG % NW
    peel = (nb - 1) + (base_g - nb) % nb
    assert base_g >= peel + 2
    rows_per_sub = N // NS
    mesh = plsc.VectorSubcoreMesh(core_axis_name="c", subcore_axis_name="s")

    @functools.partial(
        pl.kernel,
        out_type=jax.ShapeDtypeStruct((NC, N, D), jnp.float32),
        mesh=mesh,
        compiler_params=pltpu.CompilerParams(use_tc_tiling_on_sc=False),
        scratch_types=[
            pltpu.VMEM_SHARED((N, D), jnp.float32),
            pltpu.VMEM((base_g + 1, G), jnp.int32),
            pltpu.VMEM((base_g + 1, G), jnp.int32),
        ] + [pltpu.VMEM((G, D), jnp.float32)] * nb
          + [pltpu.SemaphoreType.DMA] * (2 * nb),
    )
    def agg_kernel(table_hbm, src_hbm, dst_hbm, zeros_hbm, out_hbm,
                   acc, isl, idl, *bufs):
        rows = bufs[:nb]
        sg = bufs[nb:2 * nb]
        ss = bufs[2 * nb:3 * nb]
        cid = lax.axis_index("c")
        sid = lax.axis_index("s")
        wid = sid * NC + cid
        r0 = sid * rows_per_sub
        row0 = base_g * wid + jnp.minimum(wid, rem)
        pltpu.async_copy(zeros_hbm, acc.at[pl.ds(r0, rows_per_sub)], sg[0])
        pltpu.async_copy(src_hbm.at[pl.ds(row0, base_g + 1)], isl, sg[1])
        pltpu.async_copy(dst_hbm.at[pl.ds(row0, base_g + 1)], idl, ss[0])
        pltpu.make_async_copy(zeros_hbm, acc.at[pl.ds(r0, rows_per_sub)], sg[0]).wait()
        pltpu.make_async_copy(src_hbm.at[pl.ds(row0, base_g + 1)], isl, sg[1]).wait()
        pltpu.make_async_copy(dst_hbm.at[pl.ds(row0, base_g + 1)], idl, ss[0]).wait()
        has_extra = wid < rem
        plsc.subcore_barrier()

        def start_gather(j, b):
            pltpu.async_copy(table_hbm.at[isl.at[j]], rows[b], sg[b])

        def wait_gather(j, b):
            pltpu.make_async_copy(table_hbm.at[isl.at[j]], rows[b], sg[b]).wait()

        def start_scatter(j, b):
            pltpu.async_copy(rows[b], acc.at[idl.at[j]], ss[b], add=True)

        def wait_scatter(j, b):
            pltpu.make_async_copy(rows[b], acc.at[idl.at[j]], ss[b]).wait()

        start_gather(0, 0)
        for g in range(peel):
            wait_gather(g, g % nb)
            if g >= nb - 1:
                wait_scatter(g - (nb - 1), (g + 1) % nb)
            start_gather(g + 1, (g + 1) % nb)
            start_scatter(g, g % nb)

        @pl.loop(0, (base_g - 1 - peel) // nb)
        def _(i):
            for k in range(nb):
                g = nb * i + peel + k
                b = (peel + k) % nb
                wait_gather(g, b)
                wait_scatter(g - (nb - 1), (b + 1) % nb)
                start_gather(g + 1, (b + 1) % nb)
                start_scatter(g, b)

        g = base_g - 1
        b = g % nb
        bx = (b + 1) % nb
        wait_gather(g, b)
        wait_scatter(g - (nb - 1), bx)

        @pl.when(has_extra)
        def _():
            start_gather(base_g, bx)
        start_scatter(g, b)

        @pl.when(has_extra)
        def _():
            wait_gather(base_g, bx)
            start_scatter(base_g, bx)

        for g2 in range(base_g - nb + 1, base_g):
            wait_scatter(g2, g2 % nb)

        @pl.when(has_extra)
        def _():
            wait_scatter(base_g, base_g % nb)

        plsc.subcore_barrier()
        pltpu.sync_copy(acc.at[pl.ds(r0, rows_per_sub)],
                        out_hbm.at[cid, pl.ds(r0, rows_per_sub)])

    return agg_kernel


def _dense1_call(parts, x_ext, F, wl1t, b1, wr1t, wl2t, b2, wr2t, D1, D2, BLK):
    NP = x_ext.shape[0]
    H = wl1t.shape[1]
    C = wl2t.shape[1]
    grid = NP // BLK

    def body(p_ref, x_ref, wl1_ref, b1_ref, wr1_ref, wl2_ref, b2_ref, wr2_ref,
             y2_ref, zr_ref, dinv_ref):
        p = p_ref[0] + p_ref[1]
        feat = p[:, :F]
        deg = p[:, F:F + 1]
        dinv = 1.0 / jnp.maximum(deg, 1.0)
        mean = feat * dinv
        h = jnp.dot(mean, wl1_ref[...], preferred_element_type=jnp.float32)
        h = h + b1_ref[...]
        h = h + jnp.dot(x_ref[:, :F], wr1_ref[...], preferred_element_type=jnp.float32)
        h = jnp.maximum(h, 0.0)
        y2 = jnp.dot(h, wl2_ref[...], preferred_element_type=jnp.float32)
        zr = jnp.dot(h, wr2_ref[...], preferred_element_type=jnp.float32) + b2_ref[...]
        y2_ref[...] = jnp.pad(y2, ((0, 0), (0, D2 - C)))
        zr_ref[...] = zr
        dinv_ref[...] = jnp.broadcast_to(dinv, (BLK, 8))

    full = lambda s: pl.BlockSpec(s, lambda i: (0,) * len(s))
    return pl.pallas_call(
        body,
        grid=(grid,),
        in_specs=[
            pl.BlockSpec((NC, BLK, D1), lambda i: (0, i, 0)),
            pl.BlockSpec((BLK, D1), lambda i: (i, 0)),
            full((F, H)),
            full((1, H)),
            full((F, H)),
            full((H, C)),
            full((1, C)),
            full((H, C)),
        ],
        out_specs=[
            pl.BlockSpec((BLK, D2), lambda i: (i, 0)),
            pl.BlockSpec((BLK, C), lambda i: (i, 0)),
            pl.BlockSpec((BLK, 8), lambda i: (i, 0)),
        ],
        out_shape=[
            jax.ShapeDtypeStruct((NP, D2), jnp.float32),
            jax.ShapeDtypeStruct((NP, C), jnp.float32),
            jax.ShapeDtypeStruct((NP, 8), jnp.float32),
        ],
    )(parts, x_ext, wl1t, b1, wr1t, wl2t, b2, wr2t)


def _dense2_call(parts2, zr, dinv, D2, BLK):
    NP, C = zr.shape
    grid = NP // BLK

    def body(p_ref, zr_ref, dinv_ref, o_ref):
        p = p_ref[0] + p_ref[1]
        v = p[:, :C] * dinv_ref[:, 0:1] + zr_ref[...]
        m = jnp.max(v, axis=1, keepdims=True)
        e = jnp.exp(v - m)
        s = jnp.sum(e, axis=1, keepdims=True)
        o_ref[...] = v - m - jnp.log(s)

    return pl.pallas_call(
        body,
        grid=(grid,),
        in_specs=[
            pl.BlockSpec((NC, BLK, D2), lambda i: (0, i, 0)),
            pl.BlockSpec((BLK, C), lambda i: (i, 0)),
            pl.BlockSpec((BLK, 8), lambda i: (i, 0)),
        ],
        out_specs=pl.BlockSpec((BLK, C), lambda i: (i, 0)),
        out_shape=jax.ShapeDtypeStruct((NP, C), jnp.float32),
    )(parts2, zr, dinv)


def kernel(x, edge_index, W_l1, b_l1, W_r1, W_l2, b_l2, W_r2):
    N, F = x.shape
    E = edge_index.shape[1]
    C = W_l2.shape[0]
    D1 = F + 8
    D2 = 48
    BLK = 1264
    NP = ((N + 8 * NS - 1) // (8 * NS)) * (8 * NS)
    G1, G2 = 80, 128
    assert E % G1 == 0 and E % G2 == 0
    NG1, NG2 = E // G1, E // G2

    def idx_blocks(G, NG):
        ngp = ((NG + 1 + 7) // 8) * 8
        s = jnp.pad(edge_index[0], (0, ngp * G - E)).reshape(ngp, G)
        d = jnp.pad(edge_index[1], (0, ngp * G - E)).reshape(ngp, G)
        return s, d

    src1, dst1 = idx_blocks(G1, NG1)
    src2, dst2 = (src1, dst1) if G2 == G1 else idx_blocks(G2, NG2)

    x_ext = jnp.concatenate(
        [x, jnp.ones((N, 1), x.dtype), jnp.zeros((N, D1 - F - 1), x.dtype)],
        axis=1)
    x_ext = jnp.pad(x_ext, ((0, NP - N), (0, 0)))
    zeros1 = jnp.zeros((NP // NS, D1), jnp.float32)
    zeros2 = jnp.zeros((NP // NS, D2), jnp.float32)

    agg1 = _make_edge_agg(NP, NG1, D1, G1, 2)
    parts1 = agg1(x_ext, src1, dst1, zeros1)

    y2, zr, dinv = _dense1_call(
        parts1, x_ext, F, W_l1.T, b_l1.reshape(1, -1), W_r1.T,
        W_l2.T, b_l2.reshape(1, -1), W_r2.T, D1, D2, BLK)

    agg2 = _make_edge_agg(NP, NG2, D2, G2, 3)
    parts2 = agg2(y2, src2, dst2, zeros2)

    return _dense2_call(parts2, zr, dinv, D2, BLK)[:N]
